# baseline (speedup 1.0000x reference)
import functools

import jax
import jax.numpy as jnp
from jax import lax
from jax.experimental import pallas as pl
from jax.experimental.pallas import tpu as pltpu
from jax.experimental.pallas import tpu_sc as plsc

NC, NS = 2, 16
NW = NC * NS
BB = 125
CW = 128

_MESH = plsc.VectorSubcoreMesh(
    core_axis_name="c", subcore_axis_name="s", num_cores=NC, num_subcores=NS
)


def _deg_kernel(n_nodes, n_batches):
    rows_per_tile = n_nodes // NS

    @functools.partial(
        pl.kernel,
        out_type=jax.ShapeDtypeStruct((NC, NS, rows_per_tile, CW), jnp.float32),
        mesh=_MESH,
        scratch_types=[
            pltpu.VMEM((n_batches, BB), jnp.int32),
            pltpu.VMEM((BB, CW), jnp.float32),
            pltpu.VMEM((BB, CW), jnp.float32),
            pltpu.VMEM_SHARED((n_nodes, CW), jnp.float32),
        ],
    )
    def deg(dst_hbm, ones_hbm, zeros_hbm, out_hbm, dst_v, ones_v, zeros_v, acc):
        c = lax.axis_index("c")
        s = lax.axis_index("s")
        wid = c * NS + s
        pltpu.sync_copy(dst_hbm.at[wid], dst_v)
        pltpu.sync_copy(ones_hbm, ones_v)
        pltpu.sync_copy(zeros_hbm, zeros_v)
        row0 = s * rows_per_tile
        for k in range(rows_per_tile // BB):
            pltpu.sync_copy(zeros_v, acc.at[pl.ds(row0 + k * BB, BB)])
        plsc.subcore_barrier()

        def body(j, carry):
            pltpu.sync_copy(ones_v, acc.at[dst_v.at[j]], add=True)
            return carry

        lax.fori_loop(0, n_batches, body, 0)
        plsc.subcore_barrier()
        pltpu.sync_copy(acc.at[pl.ds(row0, rows_per_tile)], out_hbm.at[c, s])

    return deg


def _seg_kernel(n_nodes, n_batches, n_chunks):
    rows_per_tile = n_nodes // NS

    @functools.partial(
        pl.kernel,
        out_type=jax.ShapeDtypeStruct(
            (NC, NS, rows_per_tile, n_chunks * CW), jnp.float32
        ),
        mesh=_MESH,
        scratch_types=[
            pltpu.VMEM((n_batches, BB), jnp.int32),
            pltpu.VMEM((n_batches, BB), jnp.int32),
            pltpu.VMEM((BB, CW), jnp.float32),
            pltpu.VMEM((BB, CW), jnp.float32),
            pltpu.SemaphoreType.DMA,
            pltpu.SemaphoreType.DMA,
            pltpu.SemaphoreType.DMA,
            pltpu.SemaphoreType.DMA,
            pltpu.VMEM_SHARED((n_nodes, CW), jnp.float32),
        ],
    )
    def seg(src_hbm, dst_hbm, zeros_hbm, *rest):
        ys = rest[:n_chunks]
        out_hbm = rest[n_chunks]
        (src_v, dst_v, buf_a, buf_b, sem_a, sem_b, sem_sa, sem_sb,
         acc) = rest[n_chunks + 1:]
        c = lax.axis_index("c")
        s = lax.axis_index("s")
        wid = c * NS + s
        pltpu.sync_copy(src_hbm.at[wid], src_v)
        pltpu.sync_copy(dst_hbm.at[wid], dst_v)
        row0 = s * rows_per_tile
        nb2 = n_batches // 2
        for ci in range(n_chunks):
            pltpu.sync_copy(zeros_hbm, buf_a)
            for k in range(rows_per_tile // BB):
                pltpu.sync_copy(buf_a, acc.at[pl.ds(row0 + k * BB, BB)])
            plsc.subcore_barrier()
            y = ys[ci]
            pltpu.async_copy(y.at[src_v.at[0]], buf_a, sem_a)
            pltpu.async_copy(y.at[src_v.at[1]], buf_b, sem_b)

            def body(j2, carry):
                j = j2 * 2
                pltpu.make_async_copy(y.at[src_v.at[j]], buf_a, sem_a).wait()
                pltpu.async_copy(buf_a, acc.at[dst_v.at[j]], sem_sa, add=True)
                pltpu.make_async_copy(y.at[src_v.at[j + 1]], buf_b, sem_b).wait()
                pltpu.async_copy(buf_b, acc.at[dst_v.at[j + 1]], sem_sb, add=True)

                @pl.when(j2 + 1 < nb2)
                def _():
                    pltpu.make_async_copy(
                        buf_a, acc.at[dst_v.at[j]], sem_sa).wait()
                    pltpu.async_copy(y.at[src_v.at[j + 2]], buf_a, sem_a)
                    pltpu.make_async_copy(
                        buf_b, acc.at[dst_v.at[j + 1]], sem_sb).wait()
                    pltpu.async_copy(y.at[src_v.at[j + 3]], buf_b, sem_b)

                return carry

            lax.fori_loop(0, nb2, body, 0)
            pltpu.make_async_copy(
                buf_a, acc.at[dst_v.at[n_batches - 2]], sem_sa).wait()
            pltpu.make_async_copy(
                buf_b, acc.at[dst_v.at[n_batches - 1]], sem_sb).wait()
            plsc.subcore_barrier()
            pltpu.sync_copy(
                acc.at[pl.ds(row0, rows_per_tile)],
                out_hbm.at[c, s, :, pl.ds(ci * CW, CW)],
            )

    return seg


def _dinv_block(deg_ref):
    deg = deg_ref[...][0, :, 0] + deg_ref[...][1, :, 0] + 1.0
    return lax.rsqrt(deg)[:, None]


def _layer1_mm(x, w1, degp, tm):
    n, in_ch = x.shape
    hid = w1.shape[1]
    n_chunks = hid // CW

    def body(x_ref, w_ref, deg_ref, *outs):
        dinv = _dinv_block(deg_ref)
        y = jnp.dot(x_ref[...], w_ref[...], preferred_element_type=jnp.float32)
        y = y * dinv
        for ci, o in enumerate(outs):
            o[...] = y[:, ci * CW:(ci + 1) * CW]

    return pl.pallas_call(
        body,
        grid=(n // tm,),
        in_specs=[
            pl.BlockSpec((tm, in_ch), lambda i: (i, 0)),
            pl.BlockSpec((in_ch, hid), lambda i: (0, 0)),
            pl.BlockSpec((NC, tm, CW), lambda i: (0, i, 0)),
        ],
        out_specs=[pl.BlockSpec((tm, CW), lambda i: (i, 0))] * n_chunks,
        out_shape=[jax.ShapeDtypeStruct((n, CW), jnp.float32)] * n_chunks,
    )(x, w1, degp)


def _layer2_mm(degp, acc1, y1c, b1, w2, tm):
    n = acc1.shape[1]
    hid = acc1.shape[2]
    out_ch = w2.shape[1]
    n_in = len(y1c)
    n_out = out_ch // CW

    def body(deg_ref, acc_ref, *refs):
        y1_refs = refs[:n_in]
        b_ref, w_ref = refs[n_in:n_in + 2]
        outs = refs[n_in + 2:]
        dinv = _dinv_block(deg_ref)
        y1 = jnp.concatenate([r[...] for r in y1_refs], axis=1)
        a = acc_ref[...]
        h = dinv * (a[0] + a[1] + y1) + b_ref[...]
        h = jnp.maximum(h, 0.0)
        y2 = jnp.dot(h, w_ref[...], preferred_element_type=jnp.float32)
        y2 = y2 * dinv
        for ci, o in enumerate(outs):
            o[...] = y2[:, ci * CW:(ci + 1) * CW]

    return pl.pallas_call(
        body,
        grid=(n // tm,),
        in_specs=[
            pl.BlockSpec((NC, tm, CW), lambda i: (0, i, 0)),
            pl.BlockSpec((NC, tm, hid), lambda i: (0, i, 0)),
        ]
        + [pl.BlockSpec((tm, CW), lambda i: (i, 0))] * n_in
        + [
            pl.BlockSpec((1, hid), lambda i: (0, 0)),
            pl.BlockSpec((hid, out_ch), lambda i: (0, 0)),
        ],
        out_specs=[pl.BlockSpec((tm, CW), lambda i: (i, 0))] * n_out,
        out_shape=[jax.ShapeDtypeStruct((n, CW), jnp.float32)] * n_out,
    )(degp, acc1, *y1c, b1.reshape(1, hid), w2)


def _final(degp, acc2, y2c, b2, tm):
    n = acc2.shape[1]
    out_ch = acc2.shape[2]
    n_in = len(y2c)

    def body(deg_ref, acc_ref, *refs):
        y2_refs = refs[:n_in]
        b_ref = refs[n_in]
        o = refs[n_in + 1]
        dinv = _dinv_block(deg_ref)
        y2 = jnp.concatenate([r[...] for r in y2_refs], axis=1)
        a = acc_ref[...]
        o[...] = dinv * (a[0] + a[1] + y2) + b_ref[...]

    return pl.pallas_call(
        body,
        grid=(n // tm,),
        in_specs=[
            pl.BlockSpec((NC, tm, CW), lambda i: (0, i, 0)),
            pl.BlockSpec((NC, tm, out_ch), lambda i: (0, i, 0)),
        ]
        + [pl.BlockSpec((tm, CW), lambda i: (i, 0))] * n_in
        + [pl.BlockSpec((1, out_ch), lambda i: (0, 0))],
        out_specs=pl.BlockSpec((tm, out_ch), lambda i: (i, 0)),
        out_shape=jax.ShapeDtypeStruct((n, out_ch), jnp.float32),
    )(degp, acc2, *y2c, b2.reshape(1, out_ch))


def kernel(x, edge_index, W1, b1, W2, b2):
    n, in_ch = x.shape
    hid = W1.shape[1]
    out_ch = W2.shape[1]
    n_edges = edge_index.shape[1]
    assert n % (NS * BB) == 0 and n_edges % (NW * BB) == 0
    n_batches = n_edges // (NW * BB)
    tm = 400
    assert n % tm == 0

    src = edge_index[0].reshape(NW, n_batches, BB)
    dst = edge_index[1].reshape(NW, n_batches, BB)
    onesc = jnp.ones((BB, CW), jnp.float32)
    zerosc = jnp.zeros((BB, CW), jnp.float32)

    degp = _deg_kernel(n, n_batches)(dst, onesc, zerosc).reshape(NC, n, CW)
    y1c = _layer1_mm(x, W1, degp, tm)
    acc1 = _seg_kernel(n, n_batches, hid // CW)(src, dst, zerosc, *y1c)
    acc1 = acc1.reshape(NC, n, hid)
    y2c = _layer2_mm(degp, acc1, y1c, b1, W2, tm)
    acc2 = _seg_kernel(n, n_batches, out_ch // CW)(src, dst, zerosc, *y2c)
    acc2 = acc2.reshape(NC, n, out_ch)
    return _final(degp, acc2, y2c, b2, tm)


# speedup vs baseline: 1.2319x; 1.2319x over previous
import functools

import jax
import jax.numpy as jnp
from jax import lax
from jax.experimental import pallas as pl
from jax.experimental.pallas import tpu as pltpu
from jax.experimental.pallas import tpu_sc as plsc

NC, NS = 2, 16
NW = NC * NS
BB = 125
CW = 128

_MESH = plsc.VectorSubcoreMesh(
    core_axis_name="c", subcore_axis_name="s", num_cores=NC, num_subcores=NS
)


def _deg_kernel(n_nodes, n_batches):
    rows_per_tile = n_nodes // NS

    @functools.partial(
        pl.kernel,
        out_type=jax.ShapeDtypeStruct((NC, NS, rows_per_tile, CW), jnp.float32),
        mesh=_MESH,
        scratch_types=[
            pltpu.VMEM((n_batches, BB), jnp.int32),
            pltpu.VMEM((BB, CW), jnp.float32),
            pltpu.VMEM((BB, CW), jnp.float32),
            pltpu.VMEM_SHARED((n_nodes, CW), jnp.float32),
        ],
    )
    def deg(dst_hbm, ones_hbm, zeros_hbm, out_hbm, dst_v, ones_v, zeros_v, acc):
        c = lax.axis_index("c")
        s = lax.axis_index("s")
        wid = c * NS + s
        pltpu.sync_copy(dst_hbm.at[wid], dst_v)
        pltpu.sync_copy(ones_hbm, ones_v)
        pltpu.sync_copy(zeros_hbm, zeros_v)
        row0 = s * rows_per_tile
        for k in range(rows_per_tile // BB):
            pltpu.sync_copy(zeros_v, acc.at[pl.ds(row0 + k * BB, BB)])
        plsc.subcore_barrier()

        def body(j, carry):
            pltpu.sync_copy(ones_v, acc.at[dst_v.at[j]], add=True)
            return carry

        lax.fori_loop(0, n_batches, body, 0)
        plsc.subcore_barrier()
        pltpu.sync_copy(acc.at[pl.ds(row0, rows_per_tile)], out_hbm.at[c, s])

    return deg


def _seg_kernel(n_nodes, n_batches, n_chunks):
    rows_per_tile = n_nodes // NS

    @functools.partial(
        pl.kernel,
        out_type=jax.ShapeDtypeStruct(
            (NC, NS, rows_per_tile, n_chunks * CW), jnp.float32
        ),
        mesh=_MESH,
        scratch_types=[
            pltpu.VMEM((n_batches, BB), jnp.int32),
            pltpu.VMEM((n_batches, BB), jnp.int32),
            pltpu.VMEM((BB, CW), jnp.float32),
            pltpu.VMEM((BB, CW), jnp.float32),
            pltpu.SemaphoreType.DMA,
            pltpu.SemaphoreType.DMA,
            pltpu.VMEM_SHARED((n_nodes, CW), jnp.float32),
        ],
    )
    def seg(src_hbm, dst_hbm, zeros_hbm, *rest):
        ys = rest[:n_chunks]
        out_hbm = rest[n_chunks]
        src_v, dst_v, buf_a, buf_b, sem_a, sem_b, acc = rest[n_chunks + 1:]
        c = lax.axis_index("c")
        s = lax.axis_index("s")
        wid = c * NS + s
        pltpu.sync_copy(src_hbm.at[wid], src_v)
        pltpu.sync_copy(dst_hbm.at[wid], dst_v)
        row0 = s * rows_per_tile
        nb2 = n_batches // 2
        for ci in range(n_chunks):
            pltpu.sync_copy(zeros_hbm, buf_a)
            for k in range(rows_per_tile // BB):
                pltpu.sync_copy(buf_a, acc.at[pl.ds(row0 + k * BB, BB)])
            plsc.subcore_barrier()
            y = ys[ci]
            pltpu.async_copy(y.at[src_v.at[0]], buf_a, sem_a)

            def body(j2, carry):
                j = j2 * 2
                pltpu.async_copy(y.at[src_v.at[j + 1]], buf_b, sem_b)
                pltpu.make_async_copy(y.at[src_v.at[j]], buf_a, sem_a).wait()
                pltpu.sync_copy(buf_a, acc.at[dst_v.at[j]], add=True)

                @pl.when(j2 + 1 < nb2)
                def _():
                    pltpu.async_copy(y.at[src_v.at[j + 2]], buf_a, sem_a)

                pltpu.make_async_copy(y.at[src_v.at[j + 1]], buf_b, sem_b).wait()
                pltpu.sync_copy(buf_b, acc.at[dst_v.at[j + 1]], add=True)
                return carry

            lax.fori_loop(0, nb2, body, 0)
            plsc.subcore_barrier()
            pltpu.sync_copy(
                acc.at[pl.ds(row0, rows_per_tile)],
                out_hbm.at[c, s, :, pl.ds(ci * CW, CW)],
            )

    return seg


def _dinv_block(deg_ref):
    deg = deg_ref[...][0, :, 0] + deg_ref[...][1, :, 0] + 1.0
    return lax.rsqrt(deg)[:, None]


def _layer1_mm(x, w1, degp, tm):
    n, in_ch = x.shape
    hid = w1.shape[1]
    n_chunks = hid // CW

    def body(x_ref, w_ref, deg_ref, *outs):
        dinv = _dinv_block(deg_ref)
        y = jnp.dot(x_ref[...], w_ref[...], preferred_element_type=jnp.float32)
        y = y * dinv
        for ci, o in enumerate(outs):
            o[...] = y[:, ci * CW:(ci + 1) * CW]

    return pl.pallas_call(
        body,
        grid=(n // tm,),
        in_specs=[
            pl.BlockSpec((tm, in_ch), lambda i: (i, 0)),
            pl.BlockSpec((in_ch, hid), lambda i: (0, 0)),
            pl.BlockSpec((NC, tm, CW), lambda i: (0, i, 0)),
        ],
        out_specs=[pl.BlockSpec((tm, CW), lambda i: (i, 0))] * n_chunks,
        out_shape=[jax.ShapeDtypeStruct((n, CW), jnp.float32)] * n_chunks,
    )(x, w1, degp)


def _layer2_mm(degp, acc1, y1c, b1, w2, tm):
    n = acc1.shape[1]
    hid = acc1.shape[2]
    out_ch = w2.shape[1]
    n_in = len(y1c)
    n_out = out_ch // CW

    def body(deg_ref, acc_ref, *refs):
        y1_refs = refs[:n_in]
        b_ref, w_ref = refs[n_in:n_in + 2]
        outs = refs[n_in + 2:]
        dinv = _dinv_block(deg_ref)
        y1 = jnp.concatenate([r[...] for r in y1_refs], axis=1)
        a = acc_ref[...]
        h = dinv * (a[0] + a[1] + y1) + b_ref[...]
        h = jnp.maximum(h, 0.0)
        y2 = jnp.dot(h, w_ref[...], preferred_element_type=jnp.float32)
        y2 = y2 * dinv
        for ci, o in enumerate(outs):
            o[...] = y2[:, ci * CW:(ci + 1) * CW]

    return pl.pallas_call(
        body,
        grid=(n // tm,),
        in_specs=[
            pl.BlockSpec((NC, tm, CW), lambda i: (0, i, 0)),
            pl.BlockSpec((NC, tm, hid), lambda i: (0, i, 0)),
        ]
        + [pl.BlockSpec((tm, CW), lambda i: (i, 0))] * n_in
        + [
            pl.BlockSpec((1, hid), lambda i: (0, 0)),
            pl.BlockSpec((hid, out_ch), lambda i: (0, 0)),
        ],
        out_specs=[pl.BlockSpec((tm, CW), lambda i: (i, 0))] * n_out,
        out_shape=[jax.ShapeDtypeStruct((n, CW), jnp.float32)] * n_out,
    )(degp, acc1, *y1c, b1.reshape(1, hid), w2)


def _final(degp, acc2, y2c, b2, tm):
    n = acc2.shape[1]
    out_ch = acc2.shape[2]
    n_in = len(y2c)

    def body(deg_ref, acc_ref, *refs):
        y2_refs = refs[:n_in]
        b_ref = refs[n_in]
        o = refs[n_in + 1]
        dinv = _dinv_block(deg_ref)
        y2 = jnp.concatenate([r[...] for r in y2_refs], axis=1)
        a = acc_ref[...]
        o[...] = dinv * (a[0] + a[1] + y2) + b_ref[...]

    return pl.pallas_call(
        body,
        grid=(n // tm,),
        in_specs=[
            pl.BlockSpec((NC, tm, CW), lambda i: (0, i, 0)),
            pl.BlockSpec((NC, tm, out_ch), lambda i: (0, i, 0)),
        ]
        + [pl.BlockSpec((tm, CW), lambda i: (i, 0))] * n_in
        + [pl.BlockSpec((1, out_ch), lambda i: (0, 0))],
        out_specs=pl.BlockSpec((tm, out_ch), lambda i: (i, 0)),
        out_shape=jax.ShapeDtypeStruct((n, out_ch), jnp.float32),
    )(degp, acc2, *y2c, b2.reshape(1, out_ch))


def kernel(x, edge_index, W1, b1, W2, b2):
    n, in_ch = x.shape
    hid = W1.shape[1]
    out_ch = W2.shape[1]
    n_edges = edge_index.shape[1]
    assert n % (NS * BB) == 0 and n_edges % (NW * BB) == 0
    n_batches = n_edges // (NW * BB)
    tm = 1000
    assert n % tm == 0

    src = edge_index[0].reshape(NW, n_batches, BB)
    dst = edge_index[1].reshape(NW, n_batches, BB)
    onesc = jnp.ones((BB, CW), jnp.float32)
    zerosc = jnp.zeros((BB, CW), jnp.float32)

    degp = _deg_kernel(n, n_batches)(dst, onesc, zerosc).reshape(NC, n, CW)
    y1c = _layer1_mm(x, W1, degp, tm)
    acc1 = _seg_kernel(n, n_batches, hid // CW)(src, dst, zerosc, *y1c)
    acc1 = acc1.reshape(NC, n, hid)
    y2c = _layer2_mm(degp, acc1, y1c, b1, W2, tm)
    acc2 = _seg_kernel(n, n_batches, out_ch // CW)(src, dst, zerosc, *y2c)
    acc2 = acc2.reshape(NC, n, out_ch)
    return _final(degp, acc2, y2c, b2, tm)


# speedup vs baseline: 1.2391x; 1.0058x over previous
import functools

import jax
import jax.numpy as jnp
from jax import lax
from jax.experimental import pallas as pl
from jax.experimental.pallas import tpu as pltpu
from jax.experimental.pallas import tpu_sc as plsc

NC, NS = 2, 16
NW = NC * NS
BB = 125
SB = 50
CW = 128

_MESH = plsc.VectorSubcoreMesh(
    core_axis_name="c", subcore_axis_name="s", num_cores=NC, num_subcores=NS
)


def _deg_kernel(n_nodes, n_batches):
    rows_per_tile = n_nodes // NS

    @functools.partial(
        pl.kernel,
        out_type=jax.ShapeDtypeStruct((NC, NS, rows_per_tile, CW), jnp.float32),
        mesh=_MESH,
        scratch_types=[
            pltpu.VMEM((n_batches, BB), jnp.int32),
            pltpu.VMEM((BB, CW), jnp.float32),
            pltpu.VMEM((BB, CW), jnp.float32),
            pltpu.VMEM_SHARED((n_nodes, CW), jnp.float32),
        ],
    )
    def deg(dst_hbm, ones_hbm, zeros_hbm, out_hbm, dst_v, ones_v, zeros_v, acc):
        c = lax.axis_index("c")
        s = lax.axis_index("s")
        wid = c * NS + s
        pltpu.sync_copy(dst_hbm.at[wid], dst_v)
        pltpu.sync_copy(ones_hbm, ones_v)
        pltpu.sync_copy(zeros_hbm, zeros_v)
        row0 = s * rows_per_tile
        for k in range(rows_per_tile // BB):
            pltpu.sync_copy(zeros_v, acc.at[pl.ds(row0 + k * BB, BB)])
        plsc.subcore_barrier()

        def body(j, carry):
            pltpu.sync_copy(ones_v, acc.at[dst_v.at[j]], add=True)
            return carry

        lax.fori_loop(0, n_batches, body, 0)
        plsc.subcore_barrier()
        pltpu.sync_copy(acc.at[pl.ds(row0, rows_per_tile)], out_hbm.at[c, s])

    return deg


def _seg_kernel(n_nodes, n_batches, n_chunks):
    rows_per_tile = n_nodes // NS
    nb3 = n_batches // 3
    tail = n_batches - nb3 * 3

    @functools.partial(
        pl.kernel,
        out_type=jax.ShapeDtypeStruct(
            (NC, NS, rows_per_tile, n_chunks * CW), jnp.float32
        ),
        mesh=_MESH,
        scratch_types=[
            pltpu.VMEM((n_batches, SB), jnp.int32),
            pltpu.VMEM((n_batches, SB), jnp.int32),
            pltpu.VMEM((SB, CW), jnp.float32),
            pltpu.VMEM((SB, CW), jnp.float32),
            pltpu.VMEM((SB, CW), jnp.float32),
            pltpu.SemaphoreType.DMA,
            pltpu.SemaphoreType.DMA,
            pltpu.SemaphoreType.DMA,
            pltpu.VMEM_SHARED((n_nodes, CW), jnp.float32),
        ],
    )
    def seg(src_hbm, dst_hbm, zeros_hbm, *rest):
        ys = rest[:n_chunks]
        out_hbm = rest[n_chunks]
        src_v, dst_v, b0, b1, b2, s0, s1, s2, acc = rest[n_chunks + 1:]
        bufs = (b0, b1, b2)
        sems = (s0, s1, s2)
        c = lax.axis_index("c")
        s = lax.axis_index("s")
        wid = c * NS + s
        pltpu.sync_copy(src_hbm.at[wid], src_v)
        pltpu.sync_copy(dst_hbm.at[wid], dst_v)
        row0 = s * rows_per_tile

        def zero_slice():
            n_full = rows_per_tile // SB
            for k in range(n_full):
                pltpu.sync_copy(b2, acc.at[pl.ds(row0 + k * SB, SB)])
            rem = rows_per_tile - n_full * SB
            if rem:
                pltpu.sync_copy(
                    b2.at[pl.ds(0, rem)],
                    acc.at[pl.ds(row0 + n_full * SB, rem)],
                )

        pltpu.async_copy(ys[0].at[src_v.at[0]], b0, s0)
        pltpu.async_copy(ys[0].at[src_v.at[1]], b1, s1)
        pltpu.sync_copy(zeros_hbm, b2)
        for ci in range(n_chunks):
            y = ys[ci]
            zero_slice()
            plsc.subcore_barrier()
            pltpu.async_copy(y.at[src_v.at[2]], b2, s2)

            def body(r, carry):
                for k in range(3):
                    j = r * 3 + k
                    buf, sem = bufs[k], sems[k]
                    pltpu.make_async_copy(y.at[src_v.at[j]], buf, sem).wait()
                    pltpu.sync_copy(buf, acc.at[dst_v.at[j]], add=True)

                    @pl.when(j + 3 < n_batches)
                    def _():
                        pltpu.async_copy(y.at[src_v.at[j + 3]], buf, sem)

                return carry

            lax.fori_loop(0, nb3, body, 0)
            for t in range(tail):
                j = nb3 * 3 + t
                buf, sem = bufs[j % 3], sems[j % 3]
                pltpu.make_async_copy(y.at[src_v.at[j]], buf, sem).wait()
                pltpu.sync_copy(buf, acc.at[dst_v.at[j]], add=True)
            plsc.subcore_barrier()
            if ci + 1 < n_chunks:
                y_next = ys[ci + 1]
                pltpu.async_copy(y_next.at[src_v.at[0]], b0, s0)
                pltpu.async_copy(y_next.at[src_v.at[1]], b1, s1)
            pltpu.sync_copy(
                acc.at[pl.ds(row0, rows_per_tile)],
                out_hbm.at[c, s, :, pl.ds(ci * CW, CW)],
            )
            if ci + 1 < n_chunks:
                pltpu.sync_copy(zeros_hbm, b2)

    return seg


def _dinv_block(deg_ref):
    deg = deg_ref[...][0, :, 0] + deg_ref[...][1, :, 0] + 1.0
    return lax.rsqrt(deg)[:, None]


def _layer1_mm(x, w1, degp, tm):
    n, in_ch = x.shape
    hid = w1.shape[1]
    n_chunks = hid // CW

    def body(x_ref, w_ref, deg_ref, *outs):
        dinv = _dinv_block(deg_ref)
        y = jnp.dot(x_ref[...], w_ref[...], preferred_element_type=jnp.float32)
        y = y * dinv
        for ci, o in enumerate(outs):
            o[...] = y[:, ci * CW:(ci + 1) * CW]

    return pl.pallas_call(
        body,
        grid=(n // tm,),
        in_specs=[
            pl.BlockSpec((tm, in_ch), lambda i: (i, 0)),
            pl.BlockSpec((in_ch, hid), lambda i: (0, 0)),
            pl.BlockSpec((NC, tm, CW), lambda i: (0, i, 0)),
        ],
        out_specs=[pl.BlockSpec((tm, CW), lambda i: (i, 0))] * n_chunks,
        out_shape=[jax.ShapeDtypeStruct((n, CW), jnp.float32)] * n_chunks,
    )(x, w1, degp)


def _layer2_mm(degp, acc1, y1c, b1, w2, tm):
    n = acc1.shape[1]
    hid = acc1.shape[2]
    out_ch = w2.shape[1]
    n_in = len(y1c)
    n_out = out_ch // CW

    def body(deg_ref, acc_ref, *refs):
        y1_refs = refs[:n_in]
        b_ref, w_ref = refs[n_in:n_in + 2]
        outs = refs[n_in + 2:]
        dinv = _dinv_block(deg_ref)
        y1 = jnp.concatenate([r[...] for r in y1_refs], axis=1)
        a = acc_ref[...]
        h = dinv * (a[0] + a[1] + y1) + b_ref[...]
        h = jnp.maximum(h, 0.0)
        y2 = jnp.dot(h, w_ref[...], preferred_element_type=jnp.float32)
        y2 = y2 * dinv
        for ci, o in enumerate(outs):
            o[...] = y2[:, ci * CW:(ci + 1) * CW]

    return pl.pallas_call(
        body,
        grid=(n // tm,),
        in_specs=[
            pl.BlockSpec((NC, tm, CW), lambda i: (0, i, 0)),
            pl.BlockSpec((NC, tm, hid), lambda i: (0, i, 0)),
        ]
        + [pl.BlockSpec((tm, CW), lambda i: (i, 0))] * n_in
        + [
            pl.BlockSpec((1, hid), lambda i: (0, 0)),
            pl.BlockSpec((hid, out_ch), lambda i: (0, 0)),
        ],
        out_specs=[pl.BlockSpec((tm, CW), lambda i: (i, 0))] * n_out,
        out_shape=[jax.ShapeDtypeStruct((n, CW), jnp.float32)] * n_out,
    )(degp, acc1, *y1c, b1.reshape(1, hid), w2)


def _final(degp, acc2, y2c, b2, tm):
    n = acc2.shape[1]
    out_ch = acc2.shape[2]
    n_in = len(y2c)

    def body(deg_ref, acc_ref, *refs):
        y2_refs = refs[:n_in]
        b_ref = refs[n_in]
        o = refs[n_in + 1]
        dinv = _dinv_block(deg_ref)
        y2 = jnp.concatenate([r[...] for r in y2_refs], axis=1)
        a = acc_ref[...]
        o[...] = dinv * (a[0] + a[1] + y2) + b_ref[...]

    return pl.pallas_call(
        body,
        grid=(n // tm,),
        in_specs=[
            pl.BlockSpec((NC, tm, CW), lambda i: (0, i, 0)),
            pl.BlockSpec((NC, tm, out_ch), lambda i: (0, i, 0)),
        ]
        + [pl.BlockSpec((tm, CW), lambda i: (i, 0))] * n_in
        + [pl.BlockSpec((1, out_ch), lambda i: (0, 0))],
        out_specs=pl.BlockSpec((tm, out_ch), lambda i: (i, 0)),
        out_shape=jax.ShapeDtypeStruct((n, out_ch), jnp.float32),
    )(degp, acc2, *y2c, b2.reshape(1, out_ch))


def kernel(x, edge_index, W1, b1, W2, b2):
    n, in_ch = x.shape
    hid = W1.shape[1]
    out_ch = W2.shape[1]
    n_edges = edge_index.shape[1]
    assert n % (NS * BB) == 0 and n_edges % (NW * BB) == 0
    assert n_edges % (NW * SB) == 0
    nb_deg = n_edges // (NW * BB)
    nb_seg = n_edges // (NW * SB)
    tm = 1000
    assert n % tm == 0

    src_s = edge_index[0].reshape(NW, nb_seg, SB)
    dst_s = edge_index[1].reshape(NW, nb_seg, SB)
    dst_d = edge_index[1].reshape(NW, nb_deg, BB)
    onesc = jnp.ones((BB, CW), jnp.float32)
    zerosc = jnp.zeros((BB, CW), jnp.float32)
    zeros_s = jnp.zeros((SB, CW), jnp.float32)

    degp = _deg_kernel(n, nb_deg)(dst_d, onesc, zerosc).reshape(NC, n, CW)
    y1c = _layer1_mm(x, W1, degp, tm)
    acc1 = _seg_kernel(n, nb_seg, hid // CW)(src_s, dst_s, zeros_s, *y1c)
    acc1 = acc1.reshape(NC, n, hid)
    y2c = _layer2_mm(degp, acc1, y1c, b1, W2, tm)
    acc2 = _seg_kernel(n, nb_seg, out_ch // CW)(src_s, dst_s, zeros_s, *y2c)
    acc2 = acc2.reshape(NC, n, out_ch)
    return _final(degp, acc2, y2c, b2, tm)


# speedup vs baseline: 1.2434x; 1.0035x over previous
import functools

import jax
import jax.numpy as jnp
from jax import lax
from jax.experimental import pallas as pl
from jax.experimental.pallas import tpu as pltpu
from jax.experimental.pallas import tpu_sc as plsc

NC, NS = 2, 16
NW = NC * NS
BB = 125
SB = 50
CW = 128

_MESH = plsc.VectorSubcoreMesh(
    core_axis_name="c", subcore_axis_name="s", num_cores=NC, num_subcores=NS
)


def _deg_kernel(n_nodes, n_batches):
    rows_per_tile = n_nodes // NS

    @functools.partial(
        pl.kernel,
        out_type=jax.ShapeDtypeStruct((NC, NS, rows_per_tile, CW), jnp.float32),
        mesh=_MESH,
        scratch_types=[
            pltpu.VMEM((n_batches, BB), jnp.int32),
            pltpu.VMEM((BB, CW), jnp.float32),
            pltpu.VMEM((BB, CW), jnp.float32),
            pltpu.VMEM_SHARED((n_nodes, CW), jnp.float32),
        ],
    )
    def deg(dst_hbm, ones_hbm, zeros_hbm, out_hbm, dst_v, ones_v, zeros_v, acc):
        c = lax.axis_index("c")
        s = lax.axis_index("s")
        wid = c * NS + s
        pltpu.sync_copy(dst_hbm.at[wid], dst_v)
        pltpu.sync_copy(ones_hbm, ones_v)
        pltpu.sync_copy(zeros_hbm, zeros_v)
        row0 = s * rows_per_tile
        for k in range(rows_per_tile // BB):
            pltpu.sync_copy(zeros_v, acc.at[pl.ds(row0 + k * BB, BB)])
        plsc.subcore_barrier()

        def body(j, carry):
            pltpu.sync_copy(ones_v, acc.at[dst_v.at[j]], add=True)
            return carry

        lax.fori_loop(0, n_batches, body, 0)
        plsc.subcore_barrier()
        pltpu.sync_copy(acc.at[pl.ds(row0, rows_per_tile)], out_hbm.at[c, s])

    return deg


def _seg_kernel(n_nodes, n_batches, n_chunks):
    rows_per_tile = n_nodes // NS
    nb3 = n_batches // 3
    tail = n_batches - nb3 * 3

    @functools.partial(
        pl.kernel,
        out_type=jax.ShapeDtypeStruct(
            (NC, NS, rows_per_tile, n_chunks * CW), jnp.float32
        ),
        mesh=_MESH,
        scratch_types=[
            pltpu.VMEM((n_batches, SB), jnp.int32),
            pltpu.VMEM((n_batches, SB), jnp.int32),
            pltpu.VMEM((SB, CW), jnp.float32),
            pltpu.VMEM((SB, CW), jnp.float32),
            pltpu.VMEM((SB, CW), jnp.float32),
            pltpu.SemaphoreType.DMA,
            pltpu.SemaphoreType.DMA,
            pltpu.SemaphoreType.DMA,
            pltpu.VMEM_SHARED((n_nodes, CW), jnp.float32),
        ],
    )
    def seg(src_hbm, dst_hbm, zeros_hbm, *rest):
        ys = rest[:n_chunks]
        out_hbm = rest[n_chunks]
        src_v, dst_v, b0, b1, b2, s0, s1, s2, acc = rest[n_chunks + 1:]
        bufs = (b0, b1, b2)
        sems = (s0, s1, s2)
        c = lax.axis_index("c")
        s = lax.axis_index("s")
        wid = c * NS + s
        pltpu.sync_copy(src_hbm.at[wid], src_v)
        pltpu.sync_copy(dst_hbm.at[wid], dst_v)
        row0 = s * rows_per_tile

        def zero_slice():
            n_full = rows_per_tile // SB
            for k in range(n_full):
                pltpu.sync_copy(b2, acc.at[pl.ds(row0 + k * SB, SB)])
            rem = rows_per_tile - n_full * SB
            if rem:
                pltpu.sync_copy(
                    b2.at[pl.ds(0, rem)],
                    acc.at[pl.ds(row0 + n_full * SB, rem)],
                )

        pltpu.async_copy(ys[0].at[src_v.at[0]], b0, s0)
        pltpu.async_copy(ys[0].at[src_v.at[1]], b1, s1)
        pltpu.sync_copy(zeros_hbm, b2)
        for ci in range(n_chunks):
            y = ys[ci]
            zero_slice()
            plsc.subcore_barrier()
            pltpu.async_copy(y.at[src_v.at[2]], b2, s2)

            def body(r, carry):
                for k in range(3):
                    j = r * 3 + k
                    buf, sem = bufs[k], sems[k]
                    pltpu.make_async_copy(y.at[src_v.at[j]], buf, sem).wait()
                    pltpu.sync_copy(buf, acc.at[dst_v.at[j]], add=True)

                    @pl.when(j + 3 < n_batches)
                    def _():
                        pltpu.async_copy(y.at[src_v.at[j + 3]], buf, sem)

                return carry

            lax.fori_loop(0, nb3, body, 0)
            for t in range(tail):
                j = nb3 * 3 + t
                buf, sem = bufs[j % 3], sems[j % 3]
                pltpu.make_async_copy(y.at[src_v.at[j]], buf, sem).wait()
                pltpu.sync_copy(buf, acc.at[dst_v.at[j]], add=True)
            plsc.subcore_barrier()
            if ci + 1 < n_chunks:
                y_next = ys[ci + 1]
                pltpu.async_copy(y_next.at[src_v.at[0]], b0, s0)
                pltpu.async_copy(y_next.at[src_v.at[1]], b1, s1)
            pltpu.sync_copy(
                acc.at[pl.ds(row0, rows_per_tile)],
                out_hbm.at[c, s, :, pl.ds(ci * CW, CW)],
            )
            if ci + 1 < n_chunks:
                pltpu.sync_copy(zeros_hbm, b2)

    return seg


def _dinv_block(deg_ref):
    deg = deg_ref[...][0, :, 0] + deg_ref[...][1, :, 0] + 1.0
    return lax.rsqrt(deg)[:, None]


def _layer1_mm(x, w1, degp, tm):
    n, in_ch = x.shape
    hid = w1.shape[1]
    n_chunks = hid // CW

    def body(x_ref, w_ref, deg_ref, *outs):
        dinv = _dinv_block(deg_ref)
        y = jnp.dot(x_ref[...], w_ref[...], preferred_element_type=jnp.float32)
        y = y * dinv
        for ci, o in enumerate(outs):
            o[...] = y[:, ci * CW:(ci + 1) * CW]

    return pl.pallas_call(
        body,
        grid=(n // tm,),
        in_specs=[
            pl.BlockSpec((tm, in_ch), lambda i: (i, 0)),
            pl.BlockSpec((in_ch, hid), lambda i: (0, 0)),
            pl.BlockSpec((NC, tm, CW), lambda i: (0, i, 0)),
        ],
        out_specs=[pl.BlockSpec((tm, CW), lambda i: (i, 0))] * n_chunks,
        out_shape=[jax.ShapeDtypeStruct((n, CW), jnp.float32)] * n_chunks,
    )(x, w1, degp)


def _layer2_mm(degp, acc1, y1c, b1, w2, tm):
    n = acc1.shape[1]
    hid = acc1.shape[2]
    out_ch = w2.shape[1]
    n_in = len(y1c)
    n_out = out_ch // CW

    def body(deg_ref, acc_ref, *refs):
        y1_refs = refs[:n_in]
        b_ref, w_ref = refs[n_in:n_in + 2]
        outs = refs[n_in + 2:]
        dinv = _dinv_block(deg_ref)
        y1 = jnp.concatenate([r[...] for r in y1_refs], axis=1)
        a = acc_ref[...]
        h = dinv * (a[0] + a[1] + y1) + b_ref[...]
        h = jnp.maximum(h, 0.0)
        y2 = jnp.dot(h, w_ref[...], preferred_element_type=jnp.float32)
        y2 = y2 * dinv
        for ci, o in enumerate(outs):
            o[...] = y2[:, ci * CW:(ci + 1) * CW]

    return pl.pallas_call(
        body,
        grid=(n // tm,),
        in_specs=[
            pl.BlockSpec((NC, tm, CW), lambda i: (0, i, 0)),
            pl.BlockSpec((NC, tm, hid), lambda i: (0, i, 0)),
        ]
        + [pl.BlockSpec((tm, CW), lambda i: (i, 0))] * n_in
        + [
            pl.BlockSpec((1, hid), lambda i: (0, 0)),
            pl.BlockSpec((hid, out_ch), lambda i: (0, 0)),
        ],
        out_specs=[pl.BlockSpec((tm, CW), lambda i: (i, 0))] * n_out,
        out_shape=[jax.ShapeDtypeStruct((n, CW), jnp.float32)] * n_out,
    )(degp, acc1, *y1c, b1.reshape(1, hid), w2)


def _final(degp, acc2, y2c, b2, tm):
    n = acc2.shape[1]
    out_ch = acc2.shape[2]
    n_in = len(y2c)

    def body(deg_ref, acc_ref, *refs):
        y2_refs = refs[:n_in]
        b_ref = refs[n_in]
        o = refs[n_in + 1]
        dinv = _dinv_block(deg_ref)
        y2 = jnp.concatenate([r[...] for r in y2_refs], axis=1)
        a = acc_ref[...]
        o[...] = dinv * (a[0] + a[1] + y2) + b_ref[...]

    return pl.pallas_call(
        body,
        grid=(n // tm,),
        in_specs=[
            pl.BlockSpec((NC, tm, CW), lambda i: (0, i, 0)),
            pl.BlockSpec((NC, tm, out_ch), lambda i: (0, i, 0)),
        ]
        + [pl.BlockSpec((tm, CW), lambda i: (i, 0))] * n_in
        + [pl.BlockSpec((1, out_ch), lambda i: (0, 0))],
        out_specs=pl.BlockSpec((tm, out_ch), lambda i: (i, 0)),
        out_shape=jax.ShapeDtypeStruct((n, out_ch), jnp.float32),
    )(degp, acc2, *y2c, b2.reshape(1, out_ch))


def kernel(x, edge_index, W1, b1, W2, b2):
    n, in_ch = x.shape
    hid = W1.shape[1]
    out_ch = W2.shape[1]
    n_edges = edge_index.shape[1]
    assert n % (NS * BB) == 0 and n_edges % (NW * BB) == 0
    assert n_edges % (NW * SB) == 0
    nb_deg = n_edges // (NW * BB)
    nb_seg = n_edges // (NW * SB)
    tm = 2000
    assert n % tm == 0

    src_s = edge_index[0].reshape(NW, nb_seg, SB)
    dst_s = edge_index[1].reshape(NW, nb_seg, SB)
    dst_d = edge_index[1].reshape(NW, nb_deg, BB)
    onesc = jnp.ones((BB, CW), jnp.float32)
    zerosc = jnp.zeros((BB, CW), jnp.float32)
    zeros_s = jnp.zeros((SB, CW), jnp.float32)

    degp = _deg_kernel(n, nb_deg)(dst_d, onesc, zerosc).reshape(NC, n, CW)
    y1c = _layer1_mm(x, W1, degp, tm)
    acc1 = _seg_kernel(n, nb_seg, hid // CW)(src_s, dst_s, zeros_s, *y1c)
    acc1 = acc1.reshape(NC, n, hid)
    y2c = _layer2_mm(degp, acc1, y1c, b1, W2, tm)
    acc2 = _seg_kernel(n, nb_seg, out_ch // CW)(src_s, dst_s, zeros_s, *y2c)
    acc2 = acc2.reshape(NC, n, out_ch)
    return _final(degp, acc2, y2c, b2, tm)


# speedup vs baseline: 1.2443x; 1.0007x over previous
import functools

import jax
import jax.numpy as jnp
from jax import lax
from jax.experimental import pallas as pl
from jax.experimental.pallas import tpu as pltpu
from jax.experimental.pallas import tpu_sc as plsc

NC, NS = 2, 16
NW = NC * NS
BB = 125
SB = 50
CW = 128

_MESH = plsc.VectorSubcoreMesh(
    core_axis_name="c", subcore_axis_name="s", num_cores=NC, num_subcores=NS
)


def _deg_kernel(n_nodes, n_batches):
    rows_per_tile = n_nodes // NS

    @functools.partial(
        pl.kernel,
        out_type=jax.ShapeDtypeStruct((NC, NS, rows_per_tile, CW), jnp.float32),
        mesh=_MESH,
        scratch_types=[
            pltpu.VMEM((n_batches, BB), jnp.int32),
            pltpu.VMEM((BB, CW), jnp.float32),
            pltpu.VMEM((BB, CW), jnp.float32),
            pltpu.VMEM_SHARED((n_nodes, CW), jnp.float32),
        ],
    )
    def deg(dst_hbm, ones_hbm, zeros_hbm, out_hbm, dst_v, ones_v, zeros_v, acc):
        c = lax.axis_index("c")
        s = lax.axis_index("s")
        wid = c * NS + s
        pltpu.sync_copy(dst_hbm.at[wid], dst_v)
        pltpu.sync_copy(ones_hbm, ones_v)
        pltpu.sync_copy(zeros_hbm, zeros_v)
        row0 = s * rows_per_tile
        for k in range(rows_per_tile // BB):
            pltpu.sync_copy(zeros_v, acc.at[pl.ds(row0 + k * BB, BB)])
        plsc.subcore_barrier()

        def body(j, carry):
            pltpu.sync_copy(ones_v, acc.at[dst_v.at[j]], add=True)
            return carry

        lax.fori_loop(0, n_batches, body, 0)
        plsc.subcore_barrier()
        pltpu.sync_copy(acc.at[pl.ds(row0, rows_per_tile)], out_hbm.at[c, s])

    return deg


def _seg_kernel(n_nodes, n_batches, n_chunks):
    rows_per_tile = n_nodes // NS
    nbuf = 3
    nb3 = n_batches // nbuf
    tail = n_batches - nb3 * nbuf

    @functools.partial(
        pl.kernel,
        out_type=jax.ShapeDtypeStruct(
            (NC, NS, rows_per_tile, n_chunks * CW), jnp.float32
        ),
        mesh=_MESH,
        scratch_types=[
            pltpu.VMEM((n_batches, SB), jnp.int32),
            pltpu.VMEM((n_batches, SB), jnp.int32),
            pltpu.VMEM((SB, CW), jnp.float32),
            pltpu.VMEM((SB, CW), jnp.float32),
            pltpu.VMEM((SB, CW), jnp.float32),
            pltpu.SemaphoreType.DMA,
            pltpu.SemaphoreType.DMA,
            pltpu.SemaphoreType.DMA,
            pltpu.VMEM_SHARED((n_nodes, CW), jnp.float32),
        ],
    )
    def seg(src_hbm, dst_hbm, zeros_hbm, *rest):
        ys = rest[:n_chunks]
        out_hbm = rest[n_chunks]
        src_v, dst_v, b0, b1, b2, s0, s1, s2, acc = rest[n_chunks + 1:]
        bufs = (b0, b1, b2)
        sems = (s0, s1, s2)
        c = lax.axis_index("c")
        s = lax.axis_index("s")
        wid = c * NS + s
        pltpu.sync_copy(src_hbm.at[wid], src_v)
        pltpu.sync_copy(dst_hbm.at[wid], dst_v)
        row0 = s * rows_per_tile

        def zero_slice():
            n_full = rows_per_tile // SB
            for k in range(n_full):
                pltpu.sync_copy(b2, acc.at[pl.ds(row0 + k * SB, SB)])
            rem = rows_per_tile - n_full * SB
            if rem:
                pltpu.sync_copy(
                    b2.at[pl.ds(0, rem)],
                    acc.at[pl.ds(row0 + n_full * SB, rem)],
                )

        pltpu.async_copy(ys[0].at[src_v.at[0]], b0, s0)
        pltpu.async_copy(ys[0].at[src_v.at[1]], b1, s1)
        pltpu.sync_copy(zeros_hbm, b2)
        for ci in range(n_chunks):
            y = ys[ci]
            zero_slice()
            plsc.subcore_barrier()
            pltpu.async_copy(y.at[src_v.at[2]], b2, s2)

            def body(r, carry):
                for k in range(nbuf):
                    j = r * nbuf + k
                    buf, sem = bufs[k], sems[k]
                    pltpu.make_async_copy(y.at[src_v.at[j]], buf, sem).wait()
                    pltpu.sync_copy(buf, acc.at[dst_v.at[j]], add=True)

                    @pl.when(j + nbuf < n_batches)
                    def _():
                        pltpu.async_copy(y.at[src_v.at[j + nbuf]], buf, sem)

                return carry

            lax.fori_loop(0, nb3, body, 0)
            for t in range(tail):
                j = nb3 * nbuf + t
                buf, sem = bufs[j % nbuf], sems[j % nbuf]
                pltpu.make_async_copy(y.at[src_v.at[j]], buf, sem).wait()
                pltpu.sync_copy(buf, acc.at[dst_v.at[j]], add=True)
            plsc.subcore_barrier()
            if ci + 1 < n_chunks:
                y_next = ys[ci + 1]
                pltpu.async_copy(y_next.at[src_v.at[0]], b0, s0)
                pltpu.async_copy(y_next.at[src_v.at[1]], b1, s1)
            pltpu.sync_copy(
                acc.at[pl.ds(row0, rows_per_tile)],
                out_hbm.at[c, s, :, pl.ds(ci * CW, CW)],
            )
            if ci + 1 < n_chunks:
                pltpu.sync_copy(zeros_hbm, b2)

    return seg


def _dinv_block(deg_ref):
    deg = deg_ref[...][0, :, 0] + deg_ref[...][1, :, 0] + 1.0
    return lax.rsqrt(deg)[:, None]


def _layer1_mm(x, w1, degp, tm):
    n, in_ch = x.shape
    hid = w1.shape[1]
    n_chunks = hid // CW

    def body(x_ref, w_ref, deg_ref, *outs):
        dinv = _dinv_block(deg_ref)
        y = jnp.dot(x_ref[...], w_ref[...], preferred_element_type=jnp.float32)
        y = y * dinv
        for ci, o in enumerate(outs):
            o[...] = y[:, ci * CW:(ci + 1) * CW]

    return pl.pallas_call(
        body,
        grid=(n // tm,),
        in_specs=[
            pl.BlockSpec((tm, in_ch), lambda i: (i, 0)),
            pl.BlockSpec((in_ch, hid), lambda i: (0, 0)),
            pl.BlockSpec((NC, tm, CW), lambda i: (0, i, 0)),
        ],
        out_specs=[pl.BlockSpec((tm, CW), lambda i: (i, 0))] * n_chunks,
        out_shape=[jax.ShapeDtypeStruct((n, CW), jnp.float32)] * n_chunks,
    )(x, w1, degp)


def _layer2_mm(degp, acc1, y1c, b1, w2, tm):
    n = acc1.shape[1]
    hid = acc1.shape[2]
    out_ch = w2.shape[1]
    n_in = len(y1c)
    n_out = out_ch // CW

    def body(deg_ref, acc_ref, *refs):
        y1_refs = refs[:n_in]
        b_ref, w_ref = refs[n_in:n_in + 2]
        outs = refs[n_in + 2:]
        dinv = _dinv_block(deg_ref)
        y1 = jnp.concatenate([r[...] for r in y1_refs], axis=1)
        a = acc_ref[...]
        h = dinv * (a[0] + a[1] + y1) + b_ref[...]
        h = jnp.maximum(h, 0.0)
        y2 = jnp.dot(h, w_ref[...], preferred_element_type=jnp.float32)
        y2 = y2 * dinv
        for ci, o in enumerate(outs):
            o[...] = y2[:, ci * CW:(ci + 1) * CW]

    return pl.pallas_call(
        body,
        grid=(n // tm,),
        in_specs=[
            pl.BlockSpec((NC, tm, CW), lambda i: (0, i, 0)),
            pl.BlockSpec((NC, tm, hid), lambda i: (0, i, 0)),
        ]
        + [pl.BlockSpec((tm, CW), lambda i: (i, 0))] * n_in
        + [
            pl.BlockSpec((1, hid), lambda i: (0, 0)),
            pl.BlockSpec((hid, out_ch), lambda i: (0, 0)),
        ],
        out_specs=[pl.BlockSpec((tm, CW), lambda i: (i, 0))] * n_out,
        out_shape=[jax.ShapeDtypeStruct((n, CW), jnp.float32)] * n_out,
    )(degp, acc1, *y1c, b1.reshape(1, hid), w2)


def _final(degp, acc2, y2c, b2, tm):
    n = acc2.shape[1]
    out_ch = acc2.shape[2]
    n_in = len(y2c)

    def body(deg_ref, acc_ref, *refs):
        y2_refs = refs[:n_in]
        b_ref = refs[n_in]
        o = refs[n_in + 1]
        dinv = _dinv_block(deg_ref)
        y2 = jnp.concatenate([r[...] for r in y2_refs], axis=1)
        a = acc_ref[...]
        o[...] = dinv * (a[0] + a[1] + y2) + b_ref[...]

    return pl.pallas_call(
        body,
        grid=(n // tm,),
        in_specs=[
            pl.BlockSpec((NC, tm, CW), lambda i: (0, i, 0)),
            pl.BlockSpec((NC, tm, out_ch), lambda i: (0, i, 0)),
        ]
        + [pl.BlockSpec((tm, CW), lambda i: (i, 0))] * n_in
        + [pl.BlockSpec((1, out_ch), lambda i: (0, 0))],
        out_specs=pl.BlockSpec((tm, out_ch), lambda i: (i, 0)),
        out_shape=jax.ShapeDtypeStruct((n, out_ch), jnp.float32),
    )(degp, acc2, *y2c, b2.reshape(1, out_ch))


def kernel(x, edge_index, W1, b1, W2, b2):
    n, in_ch = x.shape
    hid = W1.shape[1]
    out_ch = W2.shape[1]
    n_edges = edge_index.shape[1]
    assert n % (NS * BB) == 0 and n_edges % (NW * BB) == 0
    assert n_edges % (NW * SB) == 0
    nb_deg = n_edges // (NW * BB)
    nb_seg = n_edges // (NW * SB)
    tm = 2000
    assert n % tm == 0

    src_s = edge_index[0].reshape(NW, nb_seg, SB)
    dst_s = edge_index[1].reshape(NW, nb_seg, SB)
    dst_d = edge_index[1].reshape(NW, nb_deg, BB)
    onesc = jnp.ones((BB, CW), jnp.float32)
    zerosc = jnp.zeros((BB, CW), jnp.float32)
    zeros_s = jnp.zeros((SB, CW), jnp.float32)

    degp = _deg_kernel(n, nb_deg)(dst_d, onesc, zerosc).reshape(NC, n, CW)
    y1c = _layer1_mm(x, W1, degp, tm)
    acc1 = _seg_kernel(n, nb_seg, hid // CW)(src_s, dst_s, zeros_s, *y1c)
    acc1 = acc1.reshape(NC, n, hid)
    y2c = _layer2_mm(degp, acc1, y1c, b1, W2, tm)
    acc2 = _seg_kernel(n, nb_seg, out_ch // CW)(src_s, dst_s, zeros_s, *y2c)
    acc2 = acc2.reshape(NC, n, out_ch)
    return _final(degp, acc2, y2c, b2, tm)
